# baseline (speedup 1.0000x reference)
import functools

import jax
import jax.numpy as jnp
from jax import lax
from jax.experimental import pallas as pl
from jax.experimental.pallas import tpu as pltpu
from jax.experimental.pallas import tpu_sc as plsc

N_NODES = 10000
N_EDGES = 160000
D = 128

NC = 2
NS = 16
NW = NC * NS

NP = 10240
ZROW = N_NODES
PADNODE = NP - 1
EP = 163840
EPT = EP // NW
ECHUNK = 128
NCHUNK = EPT // ECHUNK

RPT = NP // NS

_mesh = plsc.VectorSubcoreMesh(core_axis_name="c", subcore_axis_name="s")
_sc_params = pltpu.CompilerParams(needs_layout_passes=False)


def _zero_f32(ref, n):
    def body(i, _):
        ref[pl.ds(i * 16, 16)] = jnp.zeros((16,), jnp.float32)
        return 0
    lax.fori_loop(0, n // 16, body, 0)


@functools.partial(
    pl.kernel,
    mesh=_mesh,
    compiler_params=_sc_params,
    out_type=[
        jax.ShapeDtypeStruct((NP,), jnp.float32),
        jax.ShapeDtypeStruct((NC, NP), jnp.float32),
        jax.ShapeDtypeStruct((2 * EP,), jnp.int32),
    ],
    scratch_types=[
        pltpu.VMEM((4, 128), jnp.int32),
        pltpu.VMEM((128,), jnp.float32),
        pltpu.VMEM((128,), jnp.int32),
        pltpu.VMEM((128,), jnp.int32),
        pltpu.VMEM((128,), jnp.float32),
        pltpu.VMEM((256,), jnp.int32),
        pltpu.VMEM((NP,), jnp.float32),
        pltpu.VMEM((RPT,), jnp.float32),
        pltpu.VMEM_SHARED((NP,), jnp.float32),
        pltpu.VMEM_SHARED((NP,), jnp.float32),
    ],
)
def _prologue(cids_hbm, src_hbm, dst_hbm, member_hbm, deg_hbm, sd2_hbm,
              cid_v, ones_v, sbuf, dbuf, msbuf, sd2buf, local_m, zrows,
              shared_m, shared_deg):
    cidx = lax.axis_index("c")
    sidx = lax.axis_index("s")
    wid = sidx * NC + cidx

    _zero_f32(zrows, RPT)
    pltpu.sync_copy(zrows, shared_m.at[pl.ds(sidx * RPT, RPT)])
    pltpu.sync_copy(zrows, shared_deg.at[pl.ds(sidx * RPT, RPT)])

    def obody(i, _):
        ones_v[pl.ds(i * 16, 16)] = jnp.ones((16,), jnp.float32)
        return 0
    lax.fori_loop(0, 128 // 16, obody, 0)
    plsc.subcore_barrier()

    pltpu.sync_copy(cids_hbm.at[pl.ds(sidx * 4, 4)], cid_v)
    for j in range(4):
        pltpu.sync_copy(ones_v, shared_m.at[cid_v.at[j]])
    plsc.subcore_barrier()

    pltpu.sync_copy(shared_m, local_m)

    ii = lax.broadcasted_iota(jnp.int32, (16,), 0)
    zv = jnp.full((16,), ZROW, jnp.int32)

    def chunk(i, _):
        base = wid * EPT + i * ECHUNK
        pltpu.sync_copy(src_hbm.at[pl.ds(base, ECHUNK)], sbuf)
        pltpu.sync_copy(dst_hbm.at[pl.ds(base, ECHUNK)], dbuf)
        for j in range(ECHUNK // 16):
            s = sbuf[pl.ds(j * 16, 16)]
            d = dbuf[pl.ds(j * 16, 16)]
            ms = plsc.load_gather(local_m, [s])
            md = plsc.load_gather(local_m, [d])
            s2 = jnp.where(md > 0.0, s, zv)
            d2 = jnp.where(ms > 0.0, d, zv)
            pos = j * 32 + 2 * ii
            plsc.store_scatter(sd2buf, [pos], s2)
            plsc.store_scatter(sd2buf, [pos + 1], d2)
            msbuf[pl.ds(j * 16, 16)] = ms
        pltpu.sync_copy(sd2buf, sd2_hbm.at[pl.ds(2 * base, 2 * ECHUNK)])
        pltpu.sync_copy(msbuf, shared_deg.at[dbuf], add=True)
        return 0
    lax.fori_loop(0, NCHUNK, chunk, 0)
    plsc.subcore_barrier()

    @pl.when(cidx == 0)
    def _():
        pltpu.sync_copy(shared_m.at[pl.ds(sidx * RPT, RPT)],
                        member_hbm.at[pl.ds(sidx * RPT, RPT)])
    pltpu.sync_copy(shared_deg.at[pl.ds(sidx * RPT, RPT)],
                    deg_hbm.at[cidx, pl.ds(sidx * RPT, RPT)])


@functools.partial(
    pl.kernel,
    mesh=_mesh,
    compiler_params=_sc_params,
    out_type=jax.ShapeDtypeStruct((NC, NP, D), jnp.float32),
    scratch_types=[
        pltpu.VMEM((128,), jnp.int32),
        pltpu.VMEM((128,), jnp.int32),
        pltpu.VMEM((ECHUNK, D), jnp.float32),
        pltpu.VMEM_SHARED((NP, D), jnp.float32),
        pltpu.SemaphoreType.DMA,
    ],
)
def _agg(xn_hbm, src_hbm, dst_hbm, agg_hbm, sbuf, dbuf, rows_v, shared_acc, sem):
    cidx = lax.axis_index("c")
    sidx = lax.axis_index("s")
    wid = sidx * NC + cidx

    def zrow(i, _):
        for j in range(D // 16):
            rows_v[i, pl.ds(j * 16, 16)] = jnp.zeros((16,), jnp.float32)
        return 0
    lax.fori_loop(0, ECHUNK, zrow, 0)
    for k in range(RPT // ECHUNK):
        pltpu.sync_copy(rows_v, shared_acc.at[pl.ds(sidx * RPT + k * ECHUNK, ECHUNK)])
    plsc.subcore_barrier()

    def chunk(i, _):
        base = wid * EPT + i * ECHUNK
        pltpu.sync_copy(src_hbm.at[pl.ds(base, ECHUNK)], sbuf)
        pltpu.sync_copy(dst_hbm.at[pl.ds(base, ECHUNK)], dbuf)
        pltpu.async_copy(xn_hbm.at[sbuf], rows_v, sem).wait()
        pltpu.sync_copy(rows_v, shared_acc.at[dbuf], add=True)
        return 0
    lax.fori_loop(0, NCHUNK, chunk, 0)
    plsc.subcore_barrier()

    pltpu.sync_copy(shared_acc.at[pl.ds(sidx * RPT, RPT)],
                    agg_hbm.at[cidx, pl.ds(sidx * RPT, RPT)])


_TPT = N_EDGES // NW
_TFULL = _TPT // ECHUNK
_TTAIL = _TPT - _TFULL * ECHUNK


@functools.partial(
    pl.kernel,
    mesh=_mesh,
    compiler_params=_sc_params,
    out_type=jax.ShapeDtypeStruct((2 * N_EDGES, D), jnp.float32),
    scratch_types=[
        pltpu.VMEM((2 * ECHUNK,), jnp.int32),
        pltpu.VMEM((2 * ECHUNK, D), jnp.float32),
        pltpu.VMEM((2 * _TTAIL,), jnp.int32),
        pltpu.VMEM((2 * _TTAIL, D), jnp.float32),
        pltpu.SemaphoreType.DMA,
        pltpu.SemaphoreType.DMA,
    ],
)
def _triples(h_hbm, sd2_hbm, out_hbm, idx_v, rows_v, tidx_v, trows_v, sem0, sem1):
    cidx = lax.axis_index("c")
    sidx = lax.axis_index("s")
    wid = sidx * NC + cidx

    def chunk(i, _):
        base = wid * _TPT + i * ECHUNK
        pltpu.sync_copy(sd2_hbm.at[pl.ds(2 * base, 2 * ECHUNK)], idx_v)
        cp0 = pltpu.async_copy(h_hbm.at[idx_v.at[pl.ds(0, 128)]],
                               rows_v.at[pl.ds(0, 128)], sem0)
        cp1 = pltpu.async_copy(h_hbm.at[idx_v.at[pl.ds(128, 128)]],
                               rows_v.at[pl.ds(128, 128)], sem1)
        cp0.wait()
        cp1.wait()
        pltpu.sync_copy(rows_v, out_hbm.at[pl.ds(2 * base, 2 * ECHUNK)])
        return 0
    lax.fori_loop(0, _TFULL, chunk, 0)

    tbase = wid * _TPT + _TFULL * ECHUNK
    pltpu.sync_copy(sd2_hbm.at[pl.ds(2 * tbase, 2 * _TTAIL)], tidx_v)
    pltpu.async_copy(h_hbm.at[tidx_v], trows_v, sem0).wait()
    pltpu.sync_copy(trows_v, out_hbm.at[pl.ds(2 * tbase, 2 * _TTAIL)])


_R = 512


def _k1_body(emb_ref, m_ref, ws_ref, wn_ref, xs_ref, xn_ref):
    x = emb_ref[...] * m_ref[...]
    xs_ref[...] = jnp.dot(x, ws_ref[...], preferred_element_type=jnp.float32)
    xn_ref[...] = jnp.dot(x, wn_ref[...], preferred_element_type=jnp.float32)


def _k2_body(xs_ref, aa_ref, ab_ref, da_ref, db_ref, m_ref, ws_ref, wn_ref,
             xs2_ref, xn2_ref):
    deg = da_ref[...] + db_ref[...]
    norm = 1.0 / jnp.maximum(deg, 1.0)
    h = jnp.maximum(xs_ref[...] + (aa_ref[...] + ab_ref[...]) * norm, 0.0)
    h = h * m_ref[...]
    xs2_ref[...] = jnp.dot(h, ws_ref[...], preferred_element_type=jnp.float32)
    xn2_ref[...] = jnp.dot(h, wn_ref[...], preferred_element_type=jnp.float32)


def _k3_body(xs_ref, aa_ref, ab_ref, da_ref, db_ref, m_ref, h_ref):
    deg = da_ref[...] + db_ref[...]
    norm = 1.0 / jnp.maximum(deg, 1.0)
    h = jnp.maximum(xs_ref[...] + (aa_ref[...] + ab_ref[...]) * norm, 0.0)
    h_ref[...] = h * m_ref[...]


_row_spec = pl.BlockSpec((_R, D), lambda i: (i, 0))
_col_spec = pl.BlockSpec((_R, 1), lambda i: (i, 0))
_w_spec = pl.BlockSpec((D, D), lambda i: (0, 0))
_grid = (NP // _R,)
_rowD = jax.ShapeDtypeStruct((NP, D), jnp.float32)

_k1 = pl.pallas_call(
    _k1_body, grid=_grid,
    in_specs=[_row_spec, _col_spec, _w_spec, _w_spec],
    out_specs=[_row_spec, _row_spec],
    out_shape=[_rowD, _rowD],
)

_k2 = pl.pallas_call(
    _k2_body, grid=_grid,
    in_specs=[_row_spec, _row_spec, _row_spec, _col_spec, _col_spec, _col_spec,
              _w_spec, _w_spec],
    out_specs=[_row_spec, _row_spec],
    out_shape=[_rowD, _rowD],
)

_k3 = pl.pallas_call(
    _k3_body, grid=_grid,
    in_specs=[_row_spec, _row_spec, _row_spec, _col_spec, _col_spec, _col_spec],
    out_specs=_row_spec,
    out_shape=_rowD,
)


def kernel(concept_ids, edge_index, concept_embedding,
           W1_self, W1_nbr, W2_self, W2_nbr):
    cids = concept_ids.reshape(64, 128)
    epad = jnp.full((EP - N_EDGES,), PADNODE, jnp.int32)
    src_p = jnp.concatenate([edge_index[0], epad])
    dst_p = jnp.concatenate([edge_index[1], epad])
    emb_p = jnp.pad(concept_embedding, ((0, NP - N_NODES), (0, 0)))

    member, deg2, sd2 = _prologue(cids, src_p, dst_p)
    m_col = member.reshape(NP, 1)
    da = deg2[0].reshape(NP, 1)
    db = deg2[1].reshape(NP, 1)

    xs1, xn1 = _k1(emb_p, m_col, W1_self, W1_nbr)
    agg1 = _agg(xn1, src_p, dst_p)
    xs2, xn2 = _k2(xs1, agg1[0], agg1[1], da, db, m_col, W2_self, W2_nbr)
    agg2 = _agg(xn2, src_p, dst_p)
    h2 = _k3(xs2, agg2[0], agg2[1], da, db, m_col)

    triples = _triples(h2, sd2)
    return triples.reshape(N_EDGES, 2 * D)


# speedup vs baseline: 1.0000x; 1.0000x over previous
import functools

import jax
import jax.numpy as jnp
from jax import lax
from jax.experimental import pallas as pl
from jax.experimental.pallas import tpu as pltpu
from jax.experimental.pallas import tpu_sc as plsc

N_NODES = 10000
N_EDGES = 160000
D = 128

NC = 2
NS = 16
NW = NC * NS

NP = 10240
ZROW = N_NODES
PADNODE = NP - 1
EP = 163840
EPT = EP // NW
ECHUNK = 128
NCHUNK = EPT // ECHUNK

RPT = NP // NS

_mesh = plsc.VectorSubcoreMesh(core_axis_name="c", subcore_axis_name="s")
_sc_params = pltpu.CompilerParams(needs_layout_passes=False)


def _zero_f32(ref, n):
    def body(i, _):
        ref[pl.ds(i * 16, 16)] = jnp.zeros((16,), jnp.float32)
        return 0
    lax.fori_loop(0, n // 16, body, 0)


@functools.partial(
    pl.kernel,
    mesh=_mesh,
    compiler_params=_sc_params,
    out_type=[
        jax.ShapeDtypeStruct((NP,), jnp.float32),
        jax.ShapeDtypeStruct((NC, NP), jnp.float32),
        jax.ShapeDtypeStruct((2 * EP,), jnp.int32),
    ],
    scratch_types=[
        pltpu.VMEM((4, 128), jnp.int32),
        pltpu.VMEM((128,), jnp.float32),
        pltpu.VMEM((128,), jnp.int32),
        pltpu.VMEM((128,), jnp.int32),
        pltpu.VMEM((128,), jnp.float32),
        pltpu.VMEM((256,), jnp.int32),
        pltpu.VMEM((NP,), jnp.float32),
        pltpu.VMEM((RPT,), jnp.float32),
        pltpu.VMEM_SHARED((NP,), jnp.float32),
        pltpu.VMEM_SHARED((NP,), jnp.float32),
    ],
)
def _prologue(cids_hbm, src_hbm, dst_hbm, member_hbm, deg_hbm, sd2_hbm,
              cid_v, ones_v, sbuf, dbuf, msbuf, sd2buf, local_m, zrows,
              shared_m, shared_deg):
    cidx = lax.axis_index("c")
    sidx = lax.axis_index("s")
    wid = sidx * NC + cidx

    _zero_f32(zrows, RPT)
    pltpu.sync_copy(zrows, shared_m.at[pl.ds(sidx * RPT, RPT)])
    pltpu.sync_copy(zrows, shared_deg.at[pl.ds(sidx * RPT, RPT)])

    def obody(i, _):
        ones_v[pl.ds(i * 16, 16)] = jnp.ones((16,), jnp.float32)
        return 0
    lax.fori_loop(0, 128 // 16, obody, 0)
    plsc.subcore_barrier()

    pltpu.sync_copy(cids_hbm.at[pl.ds(sidx * 4, 4)], cid_v)
    for j in range(4):
        pltpu.sync_copy(ones_v, shared_m.at[cid_v.at[j]])
    plsc.subcore_barrier()

    pltpu.sync_copy(shared_m, local_m)

    ii = lax.broadcasted_iota(jnp.int32, (16,), 0)
    zv = jnp.full((16,), ZROW, jnp.int32)

    def chunk(i, _):
        base = wid * EPT + i * ECHUNK
        pltpu.sync_copy(src_hbm.at[pl.ds(base, ECHUNK)], sbuf)
        pltpu.sync_copy(dst_hbm.at[pl.ds(base, ECHUNK)], dbuf)
        for j in range(ECHUNK // 16):
            s = sbuf[pl.ds(j * 16, 16)]
            d = dbuf[pl.ds(j * 16, 16)]
            ms = plsc.load_gather(local_m, [s])
            md = plsc.load_gather(local_m, [d])
            s2 = jnp.where(md > 0.0, s, zv)
            d2 = jnp.where(ms > 0.0, d, zv)
            pos = j * 32 + 2 * ii
            plsc.store_scatter(sd2buf, [pos], s2)
            plsc.store_scatter(sd2buf, [pos + 1], d2)
            msbuf[pl.ds(j * 16, 16)] = ms
        pltpu.sync_copy(sd2buf, sd2_hbm.at[pl.ds(2 * base, 2 * ECHUNK)])
        pltpu.sync_copy(msbuf, shared_deg.at[dbuf], add=True)
        return 0
    lax.fori_loop(0, NCHUNK, chunk, 0)
    plsc.subcore_barrier()

    @pl.when(cidx == 0)
    def _():
        pltpu.sync_copy(shared_m.at[pl.ds(sidx * RPT, RPT)],
                        member_hbm.at[pl.ds(sidx * RPT, RPT)])
    pltpu.sync_copy(shared_deg.at[pl.ds(sidx * RPT, RPT)],
                    deg_hbm.at[cidx, pl.ds(sidx * RPT, RPT)])


@functools.partial(
    pl.kernel,
    mesh=_mesh,
    compiler_params=_sc_params,
    out_type=jax.ShapeDtypeStruct((NC, NP, D), jnp.float32),
    scratch_types=[
        pltpu.VMEM((128,), jnp.int32),
        pltpu.VMEM((128,), jnp.int32),
        pltpu.VMEM((ECHUNK, D), jnp.float32),
        pltpu.VMEM_SHARED((NP, D), jnp.float32),
        pltpu.SemaphoreType.DMA,
    ],
)
def _agg(xn_hbm, src_hbm, dst_hbm, agg_hbm, sbuf, dbuf, rows_v, shared_acc, sem):
    cidx = lax.axis_index("c")
    sidx = lax.axis_index("s")
    wid = sidx * NC + cidx

    def zrow(i, _):
        for j in range(D // 16):
            rows_v[i, pl.ds(j * 16, 16)] = jnp.zeros((16,), jnp.float32)
        return 0
    lax.fori_loop(0, ECHUNK, zrow, 0)
    for k in range(RPT // ECHUNK):
        pltpu.sync_copy(rows_v, shared_acc.at[pl.ds(sidx * RPT + k * ECHUNK, ECHUNK)])
    plsc.subcore_barrier()

    def chunk(i, _):
        base = wid * EPT + i * ECHUNK
        pltpu.sync_copy(src_hbm.at[pl.ds(base, ECHUNK)], sbuf)
        pltpu.sync_copy(dst_hbm.at[pl.ds(base, ECHUNK)], dbuf)
        pltpu.async_copy(xn_hbm.at[sbuf], rows_v, sem).wait()
        pltpu.sync_copy(rows_v, shared_acc.at[dbuf], add=True)
        return 0
    lax.fori_loop(0, NCHUNK, chunk, 0)
    plsc.subcore_barrier()

    pltpu.sync_copy(shared_acc.at[pl.ds(sidx * RPT, RPT)],
                    agg_hbm.at[cidx, pl.ds(sidx * RPT, RPT)])


_NBLK = N_EDGES // ECHUNK
_BPT = _NBLK // NW
_BREM = _NBLK - _BPT * NW


@functools.partial(
    pl.kernel,
    mesh=_mesh,
    compiler_params=_sc_params,
    out_type=jax.ShapeDtypeStruct((2 * N_EDGES, D), jnp.float32),
    scratch_types=[
        pltpu.VMEM((2, 128), jnp.int32),
        pltpu.VMEM((2 * ECHUNK, D), jnp.float32),
        pltpu.SemaphoreType.DMA,
        pltpu.SemaphoreType.DMA,
    ],
)
def _triples(h_hbm, sd2_hbm, out_hbm, idx_v, rows_v, sem0, sem1):
    cidx = lax.axis_index("c")
    sidx = lax.axis_index("s")
    wid = sidx * NC + cidx

    def do_block(b):
        pltpu.sync_copy(sd2_hbm.at[b], idx_v)
        cp0 = pltpu.async_copy(h_hbm.at[idx_v.at[0]], rows_v.at[pl.ds(0, 128)], sem0)
        cp1 = pltpu.async_copy(h_hbm.at[idx_v.at[1]], rows_v.at[pl.ds(128, 128)], sem1)
        cp0.wait()
        cp1.wait()
        pltpu.sync_copy(rows_v, out_hbm.at[pl.ds(b * 2 * ECHUNK, 2 * ECHUNK)])

    def chunk(i, _):
        do_block(wid * _BPT + i)
        return 0
    lax.fori_loop(0, _BPT, chunk, 0)

    @pl.when(wid < _BREM)
    def _():
        do_block(NW * _BPT + wid)


_R = 512


def _k1_body(emb_ref, m_ref, ws_ref, wn_ref, xs_ref, xn_ref):
    x = emb_ref[...] * m_ref[...]
    xs_ref[...] = jnp.dot(x, ws_ref[...], preferred_element_type=jnp.float32)
    xn_ref[...] = jnp.dot(x, wn_ref[...], preferred_element_type=jnp.float32)


def _k2_body(xs_ref, aa_ref, ab_ref, da_ref, db_ref, m_ref, ws_ref, wn_ref,
             xs2_ref, xn2_ref):
    deg = da_ref[...] + db_ref[...]
    norm = 1.0 / jnp.maximum(deg, 1.0)
    h = jnp.maximum(xs_ref[...] + (aa_ref[...] + ab_ref[...]) * norm, 0.0)
    h = h * m_ref[...]
    xs2_ref[...] = jnp.dot(h, ws_ref[...], preferred_element_type=jnp.float32)
    xn2_ref[...] = jnp.dot(h, wn_ref[...], preferred_element_type=jnp.float32)


def _k3_body(xs_ref, aa_ref, ab_ref, da_ref, db_ref, m_ref, h_ref):
    deg = da_ref[...] + db_ref[...]
    norm = 1.0 / jnp.maximum(deg, 1.0)
    h = jnp.maximum(xs_ref[...] + (aa_ref[...] + ab_ref[...]) * norm, 0.0)
    h_ref[...] = h * m_ref[...]


_row_spec = pl.BlockSpec((_R, D), lambda i: (i, 0))
_col_spec = pl.BlockSpec((_R, 1), lambda i: (i, 0))
_w_spec = pl.BlockSpec((D, D), lambda i: (0, 0))
_grid = (NP // _R,)
_rowD = jax.ShapeDtypeStruct((NP, D), jnp.float32)

_k1 = pl.pallas_call(
    _k1_body, grid=_grid,
    in_specs=[_row_spec, _col_spec, _w_spec, _w_spec],
    out_specs=[_row_spec, _row_spec],
    out_shape=[_rowD, _rowD],
)

_k2 = pl.pallas_call(
    _k2_body, grid=_grid,
    in_specs=[_row_spec, _row_spec, _row_spec, _col_spec, _col_spec, _col_spec,
              _w_spec, _w_spec],
    out_specs=[_row_spec, _row_spec],
    out_shape=[_rowD, _rowD],
)

_k3 = pl.pallas_call(
    _k3_body, grid=_grid,
    in_specs=[_row_spec, _row_spec, _row_spec, _col_spec, _col_spec, _col_spec],
    out_specs=_row_spec,
    out_shape=_rowD,
)


def kernel(concept_ids, edge_index, concept_embedding,
           W1_self, W1_nbr, W2_self, W2_nbr):
    cids = concept_ids.reshape(64, 128)
    epad = jnp.full((EP - N_EDGES,), PADNODE, jnp.int32)
    src_p = jnp.concatenate([edge_index[0], epad])
    dst_p = jnp.concatenate([edge_index[1], epad])
    emb_p = jnp.pad(concept_embedding, ((0, NP - N_NODES), (0, 0)))

    member, deg2, sd2 = _prologue(cids, src_p, dst_p)
    m_col = member.reshape(NP, 1)
    da = deg2[0].reshape(NP, 1)
    db = deg2[1].reshape(NP, 1)

    xs1, xn1 = _k1(emb_p, m_col, W1_self, W1_nbr)
    agg1 = _agg(xn1, src_p, dst_p)
    xs2, xn2 = _k2(xs1, agg1[0], agg1[1], da, db, m_col, W2_self, W2_nbr)
    agg2 = _agg(xn2, src_p, dst_p)
    h2 = _k3(xs2, agg2[0], agg2[1], da, db, m_col)

    triples = _triples(h2, sd2.reshape(EP // ECHUNK, 2, 128))
    return triples.reshape(N_EDGES, 2 * D)
